# baseline (speedup 1.0000x reference)
import functools

import jax
import jax.numpy as jnp
from jax import lax
from jax.experimental import pallas as pl
from jax.experimental.pallas import tpu as pltpu
from jax.experimental.pallas import tpu_sc as plsc

_LANES = 128
_EB = 128
_NT = 16
_NC = 2
_HALVES = 2


def _ceil_to(x, m):
  return (x + m - 1) // m * m


def _make_sc_agg(n, n_pad, c_chunks, r_tile, with_deg):
  half = c_chunks // _NC
  rows_zero = n_pad // _NT

  out_type = [jax.ShapeDtypeStruct((n_pad, _LANES), jnp.float32)
              for _ in range(c_chunks)]
  if with_deg:
    out_type.append(jax.ShapeDtypeStruct((n_pad, _LANES), jnp.float32))

  r_half = r_tile // _HALVES
  scratch = [
      pltpu.VMEM((r_half, _EB), jnp.int32),
      pltpu.VMEM((r_half, _EB), jnp.int32),
      pltpu.VMEM((_EB, _LANES), jnp.float32),
      pltpu.VMEM((_EB, _LANES), jnp.float32),
      pltpu.VMEM_SHARED((n_pad, _LANES), jnp.float32),
      pltpu.SemaphoreType.DMA,
      pltpu.SemaphoreType.DMA,
      pltpu.SemaphoreType.DMA,
  ]

  mesh = plsc.VectorSubcoreMesh(core_axis_name="c", subcore_axis_name="s")

  def body(*refs):
    i = 0
    src_h, dst_h, zeros_h = refs[0:3]; i = 3
    if with_deg:
      ones_h = refs[i]; i += 1
    hs = refs[i:i + c_chunks]; i += c_chunks
    outs = refs[i:i + c_chunks]; i += c_chunks
    if with_deg:
      deg_out = refs[i]; i += 1
    src_v, dst_v, buf0, buf1 = refs[i:i + 4]; i += 4
    acc = refs[i]; gsem0, gsem1, ssem = refs[i + 1:i + 4]; i += 4

    core = lax.axis_index("c")
    sub = lax.axis_index("s")
    r_half = r_tile // _HALVES

    passes = ([(-1, deg_out)] if with_deg else []) + [
        (p, outs[p]) for p in range(c_chunks)]

    for p, out_ref in passes:
      is_deg = p < 0
      owner = 0 if is_deg else p // half

      @pl.when(core == owner)
      def _():
        off = 0
        while off < rows_zero:
          sz = min(64, rows_zero - off)
          pltpu.sync_copy(zeros_h.at[pl.ds(0, sz)],
                          acc.at[pl.ds(sub * rows_zero + off, sz)])
          off += sz
        if is_deg:
          pltpu.sync_copy(ones_h, buf0)
        plsc.subcore_barrier()

        for hf in range(_HALVES):
          base = sub * r_tile + hf * r_half
          pltpu.sync_copy(dst_h.at[pl.ds(base, r_half)], dst_v)
          if is_deg:
            def step(g, carry):
              d0 = pltpu.async_copy(buf0, acc.at[dst_v.at[2 * g]], ssem,
                                    add=True)
              d1 = pltpu.async_copy(buf0, acc.at[dst_v.at[2 * g + 1]],
                                    ssem, add=True)
              d0.wait()
              d1.wait()
              return carry
          else:
            pltpu.sync_copy(src_h.at[pl.ds(base, r_half)], src_v)
            table = hs[p]

            def step(g, carry):
              dg0 = pltpu.async_copy(table.at[src_v.at[2 * g]], buf0, gsem0)
              dg1 = pltpu.async_copy(table.at[src_v.at[2 * g + 1]], buf1,
                                     gsem1)
              dg0.wait()
              ds0 = pltpu.async_copy(buf0, acc.at[dst_v.at[2 * g]], ssem,
                                     add=True)
              dg1.wait()
              ds1 = pltpu.async_copy(buf1, acc.at[dst_v.at[2 * g + 1]],
                                     ssem, add=True)
              ds0.wait()
              ds1.wait()
              return carry

          lax.fori_loop(0, r_half // 2, step, 0)

        plsc.subcore_barrier()

        pltpu.sync_copy(acc.at[pl.ds(sub * rows_zero, rows_zero)],
                        out_ref.at[pl.ds(sub * rows_zero, rows_zero)])
        plsc.subcore_barrier()

    return None

  return pl.kernel(body, out_type=tuple(out_type), mesh=mesh,
                   scratch_types=tuple(scratch))


def _make_tc_layer(n, c_in, nb, chunked_out, h_out):
  d_in = c_in * _LANES
  c_out = h_out // _LANES
  grid = (n // nb,)

  row_spec = pl.BlockSpec((nb, _LANES), lambda i: (i, 0))
  full = lambda shape: pl.BlockSpec(shape, lambda i: tuple(0 for _ in shape))

  in_specs = ([row_spec] + [row_spec] * (2 * c_in)
              + [full((d_in, h_out)), full((d_in, h_out)), full((1, h_out))])
  if chunked_out:
    out_specs = tuple(row_spec for _ in range(c_out))
    out_shape = tuple(jax.ShapeDtypeStruct((n, _LANES), jnp.float32)
                      for _ in range(c_out))
  else:
    out_specs = pl.BlockSpec((nb, h_out), lambda i: (i, 0))
    out_shape = jax.ShapeDtypeStruct((n, h_out), jnp.float32)

  def body(*refs):
    deg_ref = refs[0]
    hs = refs[1:1 + c_in]
    aggs = refs[1 + c_in:1 + 2 * c_in]
    ws_ref, wn_ref, b_ref = refs[1 + 2 * c_in:4 + 2 * c_in]
    outs = refs[4 + 2 * c_in:]

    h = jnp.concatenate([r[...] for r in hs], axis=1)
    agg = jnp.concatenate([r[...] for r in aggs], axis=1)
    inv = 1.0 / jnp.maximum(deg_ref[:, 0:1], 1.0)
    out = (jnp.dot(h, ws_ref[...], preferred_element_type=jnp.float32)
           + jnp.dot(agg * inv, wn_ref[...], preferred_element_type=jnp.float32)
           + b_ref[...])
    out = jnp.maximum(out, 0.0)
    if chunked_out:
      for k, o in enumerate(outs):
        o[...] = out[:, k * _LANES:(k + 1) * _LANES]
    else:
      outs[0][...] = out

  return pl.pallas_call(body, grid=grid, in_specs=in_specs,
                        out_specs=out_specs, out_shape=out_shape)


@jax.jit
def kernel(x, edge_index, W_self_0, W_neigh_0, b_0, W_self_1, W_neigh_1, b_1,
           W_self_2, W_neigh_2, b_2):
  n, d_in = x.shape
  e = edge_index.shape[1]
  h_dim = W_self_0.shape[1]

  rt_total = _ceil_to(e, _EB * _NT * 8) // _EB
  r_tile = rt_total // _NT
  e_pad = rt_total * _EB
  n_pad = _ceil_to(n + 1, _NT * 8)

  perm = jnp.argsort(edge_index[0])
  src = edge_index[0][perm].astype(jnp.int32)
  dst = edge_index[1][perm].astype(jnp.int32)
  src_p = jnp.concatenate(
      [src, jnp.zeros((e_pad - e,), jnp.int32)]).reshape(rt_total, _EB)
  dst_p = jnp.concatenate(
      [dst, jnp.full((e_pad - e,), n, jnp.int32)]).reshape(rt_total, _EB)
  zeros = jnp.zeros((64, _LANES), jnp.float32)
  ones = jnp.ones((_EB, _LANES), jnp.float32)

  c0 = d_in // _LANES
  c1 = h_dim // _LANES
  xs = tuple(x[:, k * _LANES:(k + 1) * _LANES] for k in range(c0))

  sc_agg0 = _make_sc_agg(n, n_pad, c0, r_tile, with_deg=True)
  sc_agg = _make_sc_agg(n, n_pad, c1, r_tile, with_deg=False)
  tc0 = _make_tc_layer(n, c0, 1000, chunked_out=True, h_out=h_dim)
  tc1 = _make_tc_layer(n, c1, 1000, chunked_out=True, h_out=h_dim)
  tc2 = _make_tc_layer(n, c1, 1000, chunked_out=False, h_out=h_dim)

  *agg0, deg = sc_agg0(src_p, dst_p, zeros, ones, *xs)
  agg0 = tuple(a[:n] for a in agg0)
  deg = deg[:n]

  h1 = tc0(deg, *xs, *agg0, W_self_0, W_neigh_0, b_0.reshape(1, h_dim))
  agg1 = tuple(a[:n] for a in sc_agg(src_p, dst_p, zeros, *h1))
  h2 = tc1(deg, *h1, *agg1, W_self_1, W_neigh_1, b_1.reshape(1, h_dim))
  agg2 = tuple(a[:n] for a in sc_agg(src_p, dst_p, zeros, *h2))
  out = tc2(deg, *h2, *agg2, W_self_2, W_neigh_2, b_2.reshape(1, h_dim))
  return out


# speedup vs baseline: 1.1903x; 1.1903x over previous
import functools

import jax
import jax.numpy as jnp
from jax import lax
from jax.experimental import pallas as pl
from jax.experimental.pallas import tpu as pltpu
from jax.experimental.pallas import tpu_sc as plsc

_LANES = 128
_EB = 128
_NT = 16
_NC = 2
_HALVES = 2


def _ceil_to(x, m):
  return (x + m - 1) // m * m


def _make_sc_agg(n, n_pad, c_chunks, r_tile, with_deg):
  half = c_chunks // _NC
  rows_zero = n_pad // _NT

  out_type = [jax.ShapeDtypeStruct((n_pad, _LANES), jnp.float32)
              for _ in range(c_chunks)]
  if with_deg:
    out_type.append(jax.ShapeDtypeStruct((n_pad, _LANES), jnp.float32))

  r_half = r_tile // _HALVES
  scratch = [
      pltpu.VMEM((r_half, _EB), jnp.int32),
      pltpu.VMEM((r_half, _EB), jnp.int32),
      pltpu.VMEM((_EB, _LANES), jnp.float32),
      pltpu.VMEM((_EB, _LANES), jnp.float32),
      pltpu.VMEM_SHARED((n_pad, _LANES), jnp.float32),
      pltpu.SemaphoreType.DMA,
      pltpu.SemaphoreType.DMA,
      pltpu.SemaphoreType.DMA,
  ]

  mesh = plsc.VectorSubcoreMesh(core_axis_name="c", subcore_axis_name="s")

  def body(*refs):
    i = 0
    src_h, dst_h, zeros_h = refs[0:3]; i = 3
    if with_deg:
      ones_h = refs[i]; i += 1
    hs = refs[i:i + c_chunks]; i += c_chunks
    outs = refs[i:i + c_chunks]; i += c_chunks
    if with_deg:
      deg_out = refs[i]; i += 1
    src_v, dst_v, buf0, buf1 = refs[i:i + 4]; i += 4
    acc = refs[i]; gsem0, gsem1, ssem = refs[i + 1:i + 4]; i += 4

    core = lax.axis_index("c")
    sub = lax.axis_index("s")
    r_half = r_tile // _HALVES

    passes = ([(-1, deg_out)] if with_deg else []) + [
        (p, outs[p]) for p in range(c_chunks)]

    for p, out_ref in passes:
      is_deg = p < 0
      owner = 0 if is_deg else p // half

      @pl.when(core == owner)
      def _():
        off = 0
        while off < rows_zero:
          sz = min(64, rows_zero - off)
          pltpu.sync_copy(zeros_h.at[pl.ds(0, sz)],
                          acc.at[pl.ds(sub * rows_zero + off, sz)])
          off += sz
        if is_deg:
          pltpu.sync_copy(ones_h, buf0)
        plsc.subcore_barrier()

        for hf in range(_HALVES):
          base = sub * r_tile + hf * r_half
          pltpu.sync_copy(dst_h.at[pl.ds(base, r_half)], dst_v)
          if is_deg:
            def step(g, carry):
              d0 = pltpu.async_copy(buf0, acc.at[dst_v.at[2 * g]], ssem,
                                    add=True)
              d1 = pltpu.async_copy(buf0, acc.at[dst_v.at[2 * g + 1]],
                                    ssem, add=True)
              d0.wait()
              d1.wait()
              return carry
          else:
            pltpu.sync_copy(src_h.at[pl.ds(base, r_half)], src_v)
            table = hs[p]

            def step(g, carry):
              dg0 = pltpu.async_copy(table.at[src_v.at[2 * g]], buf0, gsem0)
              dg1 = pltpu.async_copy(table.at[src_v.at[2 * g + 1]], buf1,
                                     gsem1)
              dg0.wait()
              ds0 = pltpu.async_copy(buf0, acc.at[dst_v.at[2 * g]], ssem,
                                     add=True)
              dg1.wait()
              ds1 = pltpu.async_copy(buf1, acc.at[dst_v.at[2 * g + 1]],
                                     ssem, add=True)
              ds0.wait()
              ds1.wait()
              return carry

          lax.fori_loop(0, r_half // 2, step, 0)

        plsc.subcore_barrier()

        pltpu.sync_copy(acc.at[pl.ds(sub * rows_zero, rows_zero)],
                        out_ref.at[pl.ds(sub * rows_zero, rows_zero)])
        plsc.subcore_barrier()

    return None

  return pl.kernel(body, out_type=tuple(out_type), mesh=mesh,
                   scratch_types=tuple(scratch))


def _make_tc_self(n, c_in, nb, h_out):
  d_in = c_in * _LANES
  grid = (n // nb,)
  row_spec = pl.BlockSpec((nb, _LANES), lambda i: (i, 0))
  full = lambda shape: pl.BlockSpec(shape, lambda i: tuple(0 for _ in shape))

  in_specs = [row_spec] * c_in + [full((d_in, h_out)), full((1, h_out))]
  out_specs = pl.BlockSpec((nb, h_out), lambda i: (i, 0))
  out_shape = jax.ShapeDtypeStruct((n, h_out), jnp.float32)

  def body(*refs):
    hs = refs[0:c_in]
    ws_ref, b_ref, out = refs[c_in:c_in + 3]
    h = jnp.concatenate([r[...] for r in hs], axis=1)
    out[...] = (jnp.dot(h, ws_ref[...], preferred_element_type=jnp.float32)
                + b_ref[...])

  return pl.pallas_call(body, grid=grid, in_specs=in_specs,
                        out_specs=out_specs, out_shape=out_shape)


def _make_tc_comb(n, c_in, nb, chunked_out, h_out):
  d_in = c_in * _LANES
  c_out = h_out // _LANES
  grid = (n // nb,)

  row_spec = pl.BlockSpec((nb, _LANES), lambda i: (i, 0))
  full = lambda shape: pl.BlockSpec(shape, lambda i: tuple(0 for _ in shape))

  in_specs = ([pl.BlockSpec((nb, h_out), lambda i: (i, 0)), row_spec]
              + [row_spec] * c_in + [full((d_in, h_out))])
  if chunked_out:
    out_specs = tuple(row_spec for _ in range(c_out))
    out_shape = tuple(jax.ShapeDtypeStruct((n, _LANES), jnp.float32)
                      for _ in range(c_out))
  else:
    out_specs = pl.BlockSpec((nb, h_out), lambda i: (i, 0))
    out_shape = jax.ShapeDtypeStruct((n, h_out), jnp.float32)

  def body(*refs):
    self_ref, deg_ref = refs[0:2]
    aggs = refs[2:2 + c_in]
    wn_ref = refs[2 + c_in]
    outs = refs[3 + c_in:]

    agg = jnp.concatenate([r[...] for r in aggs], axis=1)
    inv = 1.0 / jnp.maximum(deg_ref[:, 0:1], 1.0)
    out = self_ref[...] + jnp.dot(agg * inv, wn_ref[...],
                                  preferred_element_type=jnp.float32)
    out = jnp.maximum(out, 0.0)
    if chunked_out:
      for k, o in enumerate(outs):
        o[...] = out[:, k * _LANES:(k + 1) * _LANES]
    else:
      outs[0][...] = out

  return pl.pallas_call(body, grid=grid, in_specs=in_specs,
                        out_specs=out_specs, out_shape=out_shape)


@jax.jit
def kernel(x, edge_index, W_self_0, W_neigh_0, b_0, W_self_1, W_neigh_1, b_1,
           W_self_2, W_neigh_2, b_2):
  n, d_in = x.shape
  e = edge_index.shape[1]
  h_dim = W_self_0.shape[1]

  rt_total = _ceil_to(e, _EB * _NT * 8) // _EB
  r_tile = rt_total // _NT
  e_pad = rt_total * _EB
  n_pad = _ceil_to(n + 1, _NT * 8)

  src = edge_index[0].astype(jnp.int32)
  dst = edge_index[1].astype(jnp.int32)
  src_p = jnp.concatenate(
      [src, jnp.zeros((e_pad - e,), jnp.int32)]).reshape(rt_total, _EB)
  dst_p = jnp.concatenate(
      [dst, jnp.full((e_pad - e,), n, jnp.int32)]).reshape(rt_total, _EB)
  zeros = jnp.zeros((64, _LANES), jnp.float32)
  ones = jnp.ones((_EB, _LANES), jnp.float32)

  c0 = d_in // _LANES
  c1 = h_dim // _LANES
  xs = tuple(x[:, k * _LANES:(k + 1) * _LANES] for k in range(c0))

  sc_agg0 = _make_sc_agg(n, n_pad, c0, r_tile, with_deg=True)
  sc_agg = _make_sc_agg(n, n_pad, c1, r_tile, with_deg=False)
  tc_self0 = _make_tc_self(n, c0, 1000, h_out=h_dim)
  tc_self1 = _make_tc_self(n, c1, 1000, h_out=h_dim)
  tc_comb0 = _make_tc_comb(n, c0, 1000, chunked_out=True, h_out=h_dim)
  tc_comb1 = _make_tc_comb(n, c1, 1000, chunked_out=True, h_out=h_dim)
  tc_comb2 = _make_tc_comb(n, c1, 1000, chunked_out=False, h_out=h_dim)

  *agg0, deg = sc_agg0(src_p, dst_p, zeros, ones, *xs)
  self0 = tc_self0(*xs, W_self_0, b_0.reshape(1, h_dim))
  agg0 = tuple(a[:n] for a in agg0)
  deg = deg[:n]
  h1 = tc_comb0(self0, deg, *agg0, W_neigh_0)

  agg1 = tuple(a[:n] for a in sc_agg(src_p, dst_p, zeros, *h1))
  self1 = tc_self1(*h1, W_self_1, b_1.reshape(1, h_dim))
  h2 = tc_comb1(self1, deg, *agg1, W_neigh_1)

  agg2 = tuple(a[:n] for a in sc_agg(src_p, dst_p, zeros, *h2))
  self2 = tc_self1(*h2, W_self_2, b_2.reshape(1, h_dim))
  out = tc_comb2(self2, deg, *agg2, W_neigh_2)
  return out
